# baseline (speedup 1.0000x reference)
import numpy as np
import jax
import jax.numpy as jnp
from jax.experimental import pallas as pl
from jax.experimental.pallas import tpu as pltpu
from jax.sharding import Mesh, PartitionSpec as P

_EPS = 1e-5


def _bilinear_matrix(n_in: int, n_out: int) -> np.ndarray:
    A = np.zeros((n_out, n_in), dtype=np.float32)
    if n_in == 1 or n_out == 1:
        A[:, 0] = 1.0
        return A
    src = np.arange(n_out, dtype=np.float64) * (n_in - 1) / (n_out - 1)
    lo = np.clip(np.floor(src).astype(np.int64), 0, n_in - 2)
    frac = (src - lo).astype(np.float32)
    A[np.arange(n_out), lo] += 1.0 - frac
    A[np.arange(n_out), lo + 1] += frac
    return A


def _mix_moments_kernel(x_ref, w_ref, z_ref, s_ref, q_ref):
    for i in range(x_ref.shape[0]):
        Xb = x_ref[i].astype(jnp.bfloat16)
        Z = jnp.dot(w_ref[...], Xb, preferred_element_type=jnp.float32)
        z_ref[i] = Z.astype(jnp.bfloat16)
        s_ref[i] = jnp.sum(Z, axis=1, keepdims=True)
        q_ref[i] = jnp.sum(Z * Z, axis=1, keepdims=True)


def _kron_upsample_kernel(z_ref, k_ref, sc_ref, sh_ref, o_ref):
    nb2, C, HW = z_ref.shape
    HoWo = k_ref.shape[1]
    z = z_ref[...].reshape(nb2 * C, HW)
    u2 = jnp.dot(z, k_ref[...], preferred_element_type=jnp.float32)
    sc = jnp.concatenate([sc_ref[...]] * nb2, axis=0)
    sh = jnp.concatenate([sh_ref[...]] * nb2, axis=0)
    o_ref[...] = (u2 * sc + sh).reshape(nb2, C, HoWo)


def kernel(x_nchw, conv_w, conv_b, bn_gamma, bn_beta):
    del conv_b
    N, C_in, H, W = x_nchw.shape
    C_out = conv_w.shape[0]
    factor = 2
    Ho, Wo = H * factor, W * factor
    cnt = N * H * W

    devs = jax.devices()
    ndev = len(devs) if N % len(devs) == 0 else 1
    Nl = N // ndev
    NB = min(4, Nl)
    NB2 = min(2, Nl)

    x = x_nchw.astype(jnp.float32).reshape(N, C_in, H * W)
    W2 = conv_w.reshape(C_out, C_in).astype(jnp.bfloat16)
    A_h = _bilinear_matrix(H, Ho)
    A_w = _bilinear_matrix(W, Wo)
    Kup = jnp.asarray(np.kron(A_h, A_w).T).astype(jnp.bfloat16)
    gamma = bn_gamma.astype(jnp.float32).reshape(C_out, 1)
    beta = bn_beta.astype(jnp.float32).reshape(C_out, 1)

    def shard_body(xs, w2, kup, g, b):
        Z, S, Q = pl.pallas_call(
            _mix_moments_kernel,
            out_shape=(jax.ShapeDtypeStruct((Nl, C_out, H * W), jnp.bfloat16),
                       jax.ShapeDtypeStruct((Nl, C_out, 1), jnp.float32),
                       jax.ShapeDtypeStruct((Nl, C_out, 1), jnp.float32)),
            grid=(Nl // NB,),
            in_specs=[pl.BlockSpec((NB, C_in, H * W), lambda n: (n, 0, 0)),
                      pl.BlockSpec((C_out, C_in), lambda n: (0, 0))],
            out_specs=(pl.BlockSpec((NB, C_out, H * W), lambda n: (n, 0, 0)),
                       pl.BlockSpec((NB, C_out, 1), lambda n: (n, 0, 0)),
                       pl.BlockSpec((NB, C_out, 1), lambda n: (n, 0, 0))),
            compiler_params=pltpu.CompilerParams(
                dimension_semantics=("arbitrary",)),
        )(xs, w2)

        s_loc = jnp.sum(S, axis=0)
        q_loc = jnp.sum(Q, axis=0)
        s_glb = jax.lax.psum(s_loc, 'd')
        q_glb = jax.lax.psum(q_loc, 'd')
        mean = s_glb / cnt
        var = jnp.maximum(q_glb / cnt - mean * mean, 0.0)
        scale = g * jax.lax.rsqrt(var + _EPS)
        shift = b - scale * mean

        flops = 2 * Nl * C_out * H * W * Ho * Wo
        bytes_accessed = Z.size * 2 + 4 * Nl * C_out * Ho * Wo
        out = pl.pallas_call(
            _kron_upsample_kernel,
            out_shape=jax.ShapeDtypeStruct((Nl, C_out, Ho * Wo), jnp.float32),
            grid=(Nl // NB2,),
            in_specs=[
                pl.BlockSpec((NB2, C_out, H * W), lambda n: (n, 0, 0)),
                pl.BlockSpec((H * W, Ho * Wo), lambda n: (0, 0)),
                pl.BlockSpec((C_out, 1), lambda n: (0, 0)),
                pl.BlockSpec((C_out, 1), lambda n: (0, 0)),
            ],
            out_specs=pl.BlockSpec((NB2, C_out, Ho * Wo),
                                   lambda n: (n, 0, 0)),
            compiler_params=pltpu.CompilerParams(
                dimension_semantics=("arbitrary",),
                vmem_limit_bytes=56 * 1024 * 1024),
            cost_estimate=pl.CostEstimate(flops=flops, transcendentals=0,
                                          bytes_accessed=bytes_accessed),
        )(Z, kup, scale, shift)
        return out

    mesh = Mesh(np.asarray(devs[:ndev]), ('d',))
    out_flat = jax.shard_map(
        shard_body, mesh=mesh,
        in_specs=(P('d'), P(), P(), P(), P()),
        out_specs=P('d'), check_vma=False)(x, W2, Kup, gamma, beta)

    return out_flat.reshape(N, C_out, Ho, Wo)


# speedup vs baseline: 2.1043x; 2.1043x over previous
import numpy as np
import jax
import jax.numpy as jnp
from jax.experimental import pallas as pl
from jax.experimental.pallas import tpu as pltpu

_EPS = 1e-5


def _bilinear_matrix(n_in: int, n_out: int) -> np.ndarray:
    A = np.zeros((n_out, n_in), dtype=np.float32)
    if n_in == 1 or n_out == 1:
        A[:, 0] = 1.0
        return A
    src = np.arange(n_out, dtype=np.float64) * (n_in - 1) / (n_out - 1)
    lo = np.clip(np.floor(src).astype(np.int64), 0, n_in - 2)
    frac = (src - lo).astype(np.float32)
    A[np.arange(n_out), lo] += 1.0 - frac
    A[np.arange(n_out), lo + 1] += frac
    return A


def _make_body(N, C_in, C_out, H, W, Ho, Wo, NB, NB2):
    P0 = N // NB
    cnt = float(N * H * W)

    def body(x_ref, w_ref, k_ref, g_ref, b_ref, o_ref, zs_ref, s_ref, q_ref):
        i = pl.program_id(0)

        @pl.when(i < P0)
        def _phase0():
            s_tot = jnp.zeros((C_out, 1), jnp.float32)
            q_tot = jnp.zeros((C_out, 1), jnp.float32)
            for k in range(NB):
                Xb = x_ref[k].astype(jnp.bfloat16)
                Z = jnp.dot(w_ref[...], Xb,
                            preferred_element_type=jnp.float32)
                zs_ref[i * NB + k] = Z.astype(jnp.bfloat16)
                s_tot += jnp.sum(Z, axis=1, keepdims=True)
                q_tot += jnp.sum(Z * Z, axis=1, keepdims=True)

            @pl.when(i == 0)
            def _init():
                s_ref[...] = s_tot
                q_ref[...] = q_tot

            @pl.when(i > 0)
            def _acc():
                s_ref[...] += s_tot
                q_ref[...] += q_tot

        @pl.when(i >= P0)
        def _phase1():
            n = i - P0
            mean = s_ref[...] / cnt
            var = jnp.maximum(q_ref[...] / cnt - mean * mean, 0.0)
            scale = g_ref[...] * jax.lax.rsqrt(var + _EPS)
            shift = b_ref[...] - scale * mean
            sc2 = jnp.concatenate([scale] * NB2, axis=0)
            sh2 = jnp.concatenate([shift] * NB2, axis=0)

            z = zs_ref[pl.ds(n * NB2, NB2)].reshape(NB2 * C_out, H * W)
            u2 = jnp.dot(z, k_ref[...],
                         preferred_element_type=jnp.float32)
            o_ref[...] = (u2 * sc2 + sh2).reshape(NB2, C_out, Ho * Wo)

    return body, P0


def kernel(x_nchw, conv_w, conv_b, bn_gamma, bn_beta):
    del conv_b
    N, C_in, H, W = x_nchw.shape
    C_out = conv_w.shape[0]
    factor = 2
    Ho, Wo = H * factor, W * factor
    NB = 4
    NB2 = 2

    x = x_nchw.astype(jnp.float32).reshape(N, C_in, H * W)
    W2 = conv_w.reshape(C_out, C_in).astype(jnp.bfloat16)
    A_h = _bilinear_matrix(H, Ho)
    A_w = _bilinear_matrix(W, Wo)
    Kup = jnp.asarray(np.kron(A_h, A_w).T).astype(jnp.bfloat16)
    gamma = bn_gamma.astype(jnp.float32).reshape(C_out, 1)
    beta = bn_beta.astype(jnp.float32).reshape(C_out, 1)

    body, P0 = _make_body(N, C_in, C_out, H, W, Ho, Wo, NB, NB2)

    flops = 2 * N * (C_out * C_in * H * W + C_out * H * W * Ho * Wo)
    bytes_accessed = 4 * x.size + 4 * N * C_out * Ho * Wo

    out_flat = pl.pallas_call(
        body,
        out_shape=jax.ShapeDtypeStruct((N, C_out, Ho * Wo), jnp.float32),
        grid=(P0 + N // NB2,),
        in_specs=[
            pl.BlockSpec((NB, C_in, H * W),
                         lambda i: (jnp.where(i < P0, i, 0), 0, 0)),
            pl.BlockSpec((C_out, C_in), lambda i: (0, 0)),
            pl.BlockSpec((H * W, Ho * Wo), lambda i: (0, 0)),
            pl.BlockSpec((C_out, 1), lambda i: (0, 0)),
            pl.BlockSpec((C_out, 1), lambda i: (0, 0)),
        ],
        out_specs=pl.BlockSpec(
            (NB2, C_out, Ho * Wo),
            lambda i: (jnp.where(i < P0, 0, i - P0), 0, 0)),
        scratch_shapes=[
            pltpu.VMEM((N, C_out, H * W), jnp.bfloat16),
            pltpu.VMEM((C_out, 1), jnp.float32),
            pltpu.VMEM((C_out, 1), jnp.float32),
        ],
        compiler_params=pltpu.CompilerParams(
            dimension_semantics=("arbitrary",),
            vmem_limit_bytes=56 * 1024 * 1024),
        cost_estimate=pl.CostEstimate(flops=flops, transcendentals=0,
                                      bytes_accessed=bytes_accessed),
    )(x, W2, Kup, gamma, beta)

    return out_flat.reshape(N, C_out, Ho, Wo)
